# baseline (speedup 1.0000x reference)
import jax
import jax.numpy as jnp
from jax.experimental import pallas as pl
from jax.experimental.pallas import tpu as pltpu

_HIDDEN = 2048
_N_EXPERTS = 64
_TOP_K = 8
_TC = 1024
_NBUF = 4
_TOKENS = 16384
_NCHUNK = _TOKENS // _TC


def _chunk_topk(x, w, bias, idx_ref, wgt_ref, t0):
    logits = jax.lax.dot_general(
        w, x, (((1,), (1,)), ((), ())), preferred_element_type=jnp.float32
    )
    scores = jax.nn.sigmoid(logits)
    sel = scores + bias

    row = jax.lax.broadcasted_iota(jnp.int32, (_N_EXPERTS, _TC), 0).astype(
        jnp.float32
    )
    row8 = jax.lax.broadcasted_iota(jnp.int32, (_TOP_K, _TC), 0)
    idx_acc = jnp.zeros((_TOP_K, _TC), jnp.float32)
    wgt_acc = jnp.zeros((_TOP_K, _TC), jnp.float32)
    neg_inf = jnp.float32(-jnp.inf)

    for k in range(_TOP_K):
        m = jnp.max(sel, axis=0, keepdims=True)
        is_max = sel == m
        idx = jnp.min(
            jnp.where(is_max, row, float(_N_EXPERTS)), axis=0, keepdims=True
        )
        onehot = row == idx
        wk = jnp.sum(jnp.where(onehot, scores, 0.0), axis=0, keepdims=True)
        idx_acc = idx_acc + jnp.where(row8 == k, idx, 0.0)
        wgt_acc = wgt_acc + jnp.where(row8 == k, wk, 0.0)
        sel = jnp.where(onehot, neg_inf, sel)

    denom = jnp.sum(wgt_acc, axis=0, keepdims=True) + 1e-20
    idx_ref[:, t0 : t0 + _TC] = idx_acc.astype(jnp.int32)
    wgt_ref[:, t0 : t0 + _TC] = wgt_acc / denom


def _router_body(x_hbm, w_ref, b_ref, idx_ref, wgt_ref, buf_ref, sem_ref):
    w = w_ref[...]
    bias = b_ref[...]

    def _copy(step, slot):
        return pltpu.make_async_copy(
            x_hbm.at[pl.ds(step * _TC, _TC), :],
            buf_ref.at[slot],
            sem_ref.at[slot],
        )

    for s in range(_NBUF):
        _copy(s, s).start()

    for step in range(_NCHUNK):
        slot = step % _NBUF
        _copy(step, slot).wait()
        x = buf_ref[slot]
        _chunk_topk(x, w, bias, idx_ref, wgt_ref, step * _TC)
        nxt = step + _NBUF
        if nxt < _NCHUNK:
            _copy(nxt, slot).start()


@jax.jit
def kernel(hidden_states, weight, e_score_correction_bias):
    batch, seq, hidden = hidden_states.shape
    tokens = batch * seq
    x = hidden_states.reshape(tokens, hidden)
    bias2d = e_score_correction_bias.reshape(_N_EXPERTS, 1)
    idx_t, wgt_t = pl.pallas_call(
        _router_body,
        in_specs=[
            pl.BlockSpec(memory_space=pl.ANY),
            pl.BlockSpec((_N_EXPERTS, hidden), lambda: (0, 0)),
            pl.BlockSpec((_N_EXPERTS, 1), lambda: (0, 0)),
        ],
        out_specs=[
            pl.BlockSpec((_TOP_K, tokens), lambda: (0, 0)),
            pl.BlockSpec((_TOP_K, tokens), lambda: (0, 0)),
        ],
        out_shape=[
            jax.ShapeDtypeStruct((_TOP_K, tokens), jnp.int32),
            jax.ShapeDtypeStruct((_TOP_K, tokens), jnp.float32),
        ],
        scratch_shapes=[
            pltpu.VMEM((_NBUF, _TC, _HIDDEN), jnp.float32),
            pltpu.SemaphoreType.DMA((_NBUF,)),
        ],
    )(x, weight, bias2d)
    return idx_t.T, wgt_t.T
